# Initial kernel scaffold; baseline (speedup 1.0000x reference)
import jax
import jax.numpy as jnp
from jax.experimental import pallas as pl


def kernel(edge_inv_global, edge_index, edge_length, a, pos, pos_perturbed, node2graph, is_sidechain, log):
    raise NotImplementedError("write your pallas kernel here")


# speedup vs baseline: 72.4941x; 72.4941x over previous
import functools

import jax
import jax.numpy as jnp
from jax import lax
from jax.experimental import pallas as pl
from jax.experimental.pallas import tpu as pltpu
from jax.experimental.pallas import tpu_sc as plsc

_NC = 2
_NS = 16
_NW = _NC * _NS
_L = 16
_CH = 128

_MESH = plsc.VectorSubcoreMesh(
    core_axis_name="c", subcore_axis_name="s", num_cores=_NC, num_subcores=_NS
)

_SC_PARAMS = pltpu.CompilerParams(
    needs_layout_passes=False, use_tc_tiling_on_sc=False
)


def _rsqrt(x):
    i = plsc.bitcast(x, jnp.int32)
    i = jnp.int32(0x5F3759DF) - lax.shift_right_logical(i, 1)
    y = plsc.bitcast(i, jnp.float32)
    for _ in range(3):
        y = y * (1.5 - 0.5 * x * y * y)
    return y


def _iota():
    return jnp.arange(_L, dtype=jnp.int32)


def _fill(k):
    return jnp.full((_L,), k, dtype=jnp.int32)


def _make_pack_kernel(npad, g_sz):
    npt = npad // _NW
    grp = npt // _L

    @functools.partial(
        pl.kernel,
        out_type=jax.ShapeDtypeStruct((npad, 8), jnp.float32),
        mesh=_MESH,
        compiler_params=_SC_PARAMS,
        scratch_types=[
            pltpu.VMEM((g_sz,), jnp.float32),
            pltpu.VMEM((npt,), jnp.int32),
            pltpu.VMEM((npt,), jnp.int32),
            pltpu.VMEM((npt, 3), jnp.float32),
            pltpu.VMEM((npt, 3), jnp.float32),
            pltpu.VMEM((npt, 8), jnp.float32),
        ],
    )
    def pack(a_hbm, n2g_hbm, side_hbm, pos_hbm, q_hbm, out_hbm,
             ctab, n2g, side, posb, qb, packb):
        wid = lax.axis_index("c") * _NS + lax.axis_index("s")
        base = wid * npt
        pltpu.sync_copy(a_hbm, ctab)
        pltpu.sync_copy(n2g_hbm.at[pl.ds(base, npt)], n2g)
        pltpu.sync_copy(side_hbm.at[pl.ds(base, npt)], side)
        pltpu.sync_copy(pos_hbm.at[pl.ds(base, npt)], posb)
        pltpu.sync_copy(q_hbm.at[pl.ds(base, npt)], qb)
        for i in range(g_sz // _L):
            av = ctab[pl.ds(i * _L, _L)]
            r = av / (1.0 - av)
            ctab[pl.ds(i * _L, _L)] = r * _rsqrt(r)

        def body(g, carry):
            row = _iota() + g * _L
            gidx = plsc.load_gather(n2g, [row])
            sidv = plsc.load_gather(side, [row]).astype(jnp.float32)
            cv = plsc.load_gather(ctab, [gidx])
            for k in range(3):
                plsc.store_scatter(packb, [row, _fill(k)],
                                   plsc.load_gather(posb, [row, _fill(k)]))
                plsc.store_scatter(packb, [row, _fill(3 + k)],
                                   plsc.load_gather(qb, [row, _fill(k)]))
            plsc.store_scatter(packb, [row, _fill(6)], cv)
            plsc.store_scatter(packb, [row, _fill(7)], sidv)
            return carry

        lax.fori_loop(0, grp, body, 0)
        pltpu.sync_copy(packb, out_hbm.at[pl.ds(base, npt)])

    return pack


def _make_edge_kernel(npad, epad):
    ept = epad // _NW
    nchunks = ept // _CH
    zrows = npad // _NS

    @functools.partial(
        pl.kernel,
        out_type=jax.ShapeDtypeStruct((_NC * npad, 8), jnp.float32),
        mesh=_MESH,
        compiler_params=_SC_PARAMS,
        scratch_types=[
            pltpu.VMEM((2, _CH), jnp.int32),
            pltpu.VMEM((_CH,), jnp.float32),
            pltpu.VMEM((_CH,), jnp.float32),
            pltpu.VMEM((_CH, 8), jnp.float32),
            pltpu.VMEM((_CH, 8), jnp.float32),
            pltpu.VMEM((_CH, 8), jnp.float32),
            pltpu.VMEM((_CH, 8), jnp.float32),
            pltpu.VMEM_SHARED((npad, 8), jnp.float32),
            pltpu.SemaphoreType.DMA,
            pltpu.SemaphoreType.DMA,
        ],
    )
    def edge(src_hbm, dst_hbm, el_hbm, ei_hbm, packed_hbm, zeros_hbm, out_hbm,
             idx2, elb, eib, rs, rd, cs, cd, acc, sem1, sem2):
        cid = lax.axis_index("c")
        sid = lax.axis_index("s")
        wid = cid * _NS + sid
        pltpu.sync_copy(zeros_hbm.at[pl.ds(sid * zrows, zrows)],
                        acc.at[pl.ds(sid * zrows, zrows)])
        zv = jnp.zeros((_L,), jnp.float32)
        for g0 in range(_CH // _L):
            r0 = _iota() + g0 * _L
            for k0 in range(3, 8):
                plsc.store_scatter(cs, [r0, _fill(k0)], zv)
                plsc.store_scatter(cd, [r0, _fill(k0)], zv)
        plsc.subcore_barrier()
        tbase = wid * ept

        def body(t, carry):
            eb = pl.multiple_of(tbase + t * _CH, _CH)
            pltpu.sync_copy(src_hbm.at[pl.ds(eb, _CH)], idx2.at[0])
            pltpu.sync_copy(dst_hbm.at[pl.ds(eb, _CH)], idx2.at[1])
            pltpu.sync_copy(el_hbm.at[pl.ds(eb, _CH)], elb)
            pltpu.sync_copy(ei_hbm.at[pl.ds(eb, _CH)], eib)
            g1 = pltpu.async_copy(packed_hbm.at[idx2.at[0]], rs, sem1)
            g2 = pltpu.async_copy(packed_hbm.at[idx2.at[1]], rd, sem2)
            g1.wait()
            g2.wait()
            for g in range(_CH // _L):
                row = _iota() + g * _L
                sv = [plsc.load_gather(rs, [row, _fill(k)]) for k in range(8)]
                dv = [plsc.load_gather(rd, [row, _fill(k)]) for k in range(8)]
                el_v = elb[pl.ds(g * _L, _L)]
                ei_v = eib[pl.ds(g * _L, _L)]
                dx = sv[0] - dv[0]
                dy = sv[1] - dv[1]
                dz = sv[2] - dv[2]
                d2 = dx * dx + dy * dy + dz * dz
                dgt = d2 * _rsqrt(jnp.maximum(d2, 1e-30))
                is_train = (sv[7] + dv[7]) > 0.0
                dtar = jnp.where(is_train, (dgt - el_v) * sv[6], 0.0)
                w = (ei_v - dtar) / el_v
                vx = w * (sv[3] - dv[3])
                vy = w * (sv[4] - dv[4])
                vz = w * (sv[5] - dv[5])
                plsc.store_scatter(cs, [row, _fill(0)], vx)
                plsc.store_scatter(cs, [row, _fill(1)], vy)
                plsc.store_scatter(cs, [row, _fill(2)], vz)
                plsc.store_scatter(cd, [row, _fill(0)], -vx)
                plsc.store_scatter(cd, [row, _fill(1)], -vy)
                plsc.store_scatter(cd, [row, _fill(2)], -vz)
            pltpu.sync_copy(cs, acc.at[idx2.at[0]], add=True)
            pltpu.sync_copy(cd, acc.at[idx2.at[1]], add=True)
            return carry

        lax.fori_loop(0, nchunks, body, 0)
        plsc.subcore_barrier()
        pltpu.sync_copy(acc.at[pl.ds(sid * zrows, zrows)],
                        out_hbm.at[pl.ds(cid * npad + sid * zrows, zrows)])

    return edge


def _make_reduce_kernel(n_real):
    scale = 10.0 / (3.0 * n_real)

    def body(x_ref, o_ref):
        x = x_ref[...]
        s = x[0] + x[1]
        o_ref[0, 0] = jnp.sum(s * s) * scale

    return functools.partial(
        pl.pallas_call,
        body,
        out_shape=jax.ShapeDtypeStruct((1, 1), jnp.float32),
        out_specs=pl.BlockSpec(memory_space=pltpu.SMEM),
    )()


def kernel(edge_inv_global, edge_index, edge_length, a, pos, pos_perturbed,
           node2graph, is_sidechain, log):
    n = pos.shape[0]
    e = edge_index.shape[1]
    g_sz = a.shape[0]

    quant_n = _NW * _L * 2
    npad = ((n + quant_n - 1) // quant_n) * quant_n
    quant_e = _NW * _CH
    epad = ((e + quant_e - 1) // quant_e) * quant_e

    src = jnp.pad(edge_index[0], (0, epad - e))
    dst = jnp.pad(edge_index[1], (0, epad - e))
    el = jnp.pad(edge_length[:, 0], (0, epad - e), constant_values=1.0)
    ei = jnp.pad(edge_inv_global[:, 0], (0, epad - e))

    posp = jnp.pad(pos, ((0, npad - n), (0, 0)))
    qp = jnp.pad(pos_perturbed, ((0, npad - n), (0, 0)))
    n2gp = jnp.pad(node2graph, (0, npad - n))
    sidep = jnp.pad(is_sidechain.astype(jnp.int32), (0, npad - n))
    zeros = jnp.zeros((npad, 8), jnp.float32)

    packed = _make_pack_kernel(npad, g_sz)(a, n2gp, sidep, posp, qp)
    partials = _make_edge_kernel(npad, epad)(src, dst, el, ei, packed, zeros)
    red = _make_reduce_kernel(n)(partials.reshape(_NC, npad * 8 // 128, 128))
    return red[0, 0]


# speedup vs baseline: 195.4802x; 2.6965x over previous
import functools

import jax
import jax.numpy as jnp
from jax import lax
from jax.experimental import pallas as pl
from jax.experimental.pallas import tpu as pltpu
from jax.experimental.pallas import tpu_sc as plsc

_NC = 2
_NS = 16
_NW = _NC * _NS
_L = 16
_CH = 128

_MESH = plsc.VectorSubcoreMesh(
    core_axis_name="c", subcore_axis_name="s", num_cores=_NC, num_subcores=_NS
)

_SC_PARAMS = pltpu.CompilerParams(
    needs_layout_passes=False, use_tc_tiling_on_sc=False
)


def _rsqrt(x):
    i = plsc.bitcast(x, jnp.int32)
    i = jnp.int32(0x5F3759DF) - lax.shift_right_logical(i, 1)
    y = plsc.bitcast(i, jnp.float32)
    for _ in range(3):
        y = y * (1.5 - 0.5 * x * y * y)
    return y


def _iota():
    return jnp.arange(_L, dtype=jnp.int32)


def _fill(k):
    return jnp.full((_L,), k, dtype=jnp.int32)


def _make_pack_kernel(npad, g_sz):
    npt = npad // _NW
    grp = npt // _L

    @functools.partial(
        pl.kernel,
        out_type=jax.ShapeDtypeStruct((npad, 8), jnp.float32),
        mesh=_MESH,
        compiler_params=_SC_PARAMS,
        scratch_types=[
            pltpu.VMEM((g_sz,), jnp.float32),
            pltpu.VMEM((npt,), jnp.int32),
            pltpu.VMEM((npt,), jnp.int32),
            pltpu.VMEM((npt, 3), jnp.float32),
            pltpu.VMEM((npt, 3), jnp.float32),
            pltpu.VMEM((npt, 8), jnp.float32),
        ],
    )
    def pack(a_hbm, n2g_hbm, side_hbm, pos_hbm, q_hbm, out_hbm,
             ctab, n2g, side, posb, qb, packb):
        wid = lax.axis_index("c") * _NS + lax.axis_index("s")
        base = wid * npt
        pltpu.sync_copy(a_hbm, ctab)
        pltpu.sync_copy(n2g_hbm.at[pl.ds(base, npt)], n2g)
        pltpu.sync_copy(side_hbm.at[pl.ds(base, npt)], side)
        pltpu.sync_copy(pos_hbm.at[pl.ds(base, npt)], posb)
        pltpu.sync_copy(q_hbm.at[pl.ds(base, npt)], qb)
        for i in range(g_sz // _L):
            av = ctab[pl.ds(i * _L, _L)]
            r = av / (1.0 - av)
            ctab[pl.ds(i * _L, _L)] = r * _rsqrt(r)

        def body(g, carry):
            row = _iota() + g * _L
            gidx = plsc.load_gather(n2g, [row])
            sidv = plsc.load_gather(side, [row]).astype(jnp.float32)
            cv = plsc.load_gather(ctab, [gidx])
            for k in range(3):
                plsc.store_scatter(packb, [row, _fill(k)],
                                   plsc.load_gather(posb, [row, _fill(k)]))
                plsc.store_scatter(packb, [row, _fill(3 + k)],
                                   plsc.load_gather(qb, [row, _fill(k)]))
            plsc.store_scatter(packb, [row, _fill(6)], cv)
            plsc.store_scatter(packb, [row, _fill(7)], sidv)
            return carry

        lax.fori_loop(0, grp, body, 0)
        pltpu.sync_copy(packb, out_hbm.at[pl.ds(base, npt)])

    return pack


def _make_edge_kernel(npad, epad):
    ept = epad // _NW
    nchunks = ept // _CH
    zrows = npad // _NS

    assert nchunks >= 4 and nchunks % 2 == 0

    @functools.partial(
        pl.kernel,
        out_type=jax.ShapeDtypeStruct((_NC * npad, 8), jnp.float32),
        mesh=_MESH,
        compiler_params=_SC_PARAMS,
        scratch_types=[
            pltpu.VMEM((2, 2, _CH), jnp.int32),
            pltpu.VMEM((2, 2, _CH), jnp.int32),
            pltpu.VMEM((2, _CH), jnp.float32),
            pltpu.VMEM((2, _CH), jnp.float32),
            pltpu.VMEM((2, _CH, 8), jnp.float32),
            pltpu.VMEM((2, _CH, 8), jnp.float32),
            pltpu.VMEM((2, _CH, 8), jnp.float32),
            pltpu.VMEM((2, _CH, 8), jnp.float32),
            pltpu.VMEM_SHARED((npad, 8), jnp.float32),
            pltpu.SemaphoreType.DMA,
            pltpu.SemaphoreType.DMA,
            pltpu.SemaphoreType.DMA,
            pltpu.SemaphoreType.DMA,
            pltpu.SemaphoreType.DMA,
            pltpu.SemaphoreType.DMA,
        ],
    )
    def edge(src_hbm, dst_hbm, el_hbm, ei_hbm, packed_hbm, zeros_hbm, out_hbm,
             idxg, idxs, elb, eib, rs, rd, cs, cd, acc,
             sl0, sl1, sg0, sg1, ss0, ss1):
        cid = lax.axis_index("c")
        sid = lax.axis_index("s")
        wid = cid * _NS + sid
        sem_lin = (sl0, sl1)
        sem_g = (sg0, sg1)
        sem_s = (ss0, ss1)
        tbase = wid * ept

        def ebase(t):
            return pl.multiple_of(tbase + t * _CH, _CH)

        def lin_issue(t, b):
            eb = ebase(t)
            pltpu.async_copy(src_hbm.at[pl.ds(eb, _CH)], idxg.at[b, 0],
                             sem_lin[b])
            pltpu.async_copy(dst_hbm.at[pl.ds(eb, _CH)], idxg.at[b, 1],
                             sem_lin[b])
            pltpu.async_copy(el_hbm.at[pl.ds(eb, _CH)], elb.at[b], sem_lin[b])
            pltpu.async_copy(ei_hbm.at[pl.ds(eb, _CH)], eib.at[b], sem_lin[b])

        def lin_wait(b):
            z = pl.ds(0, _CH)
            pltpu.make_async_copy(src_hbm.at[z], idxg.at[b, 0],
                                  sem_lin[b]).wait()
            pltpu.make_async_copy(dst_hbm.at[z], idxg.at[b, 1],
                                  sem_lin[b]).wait()
            pltpu.make_async_copy(el_hbm.at[z], elb.at[b], sem_lin[b]).wait()
            pltpu.make_async_copy(ei_hbm.at[z], eib.at[b], sem_lin[b]).wait()

        def gat_issue(b):
            pltpu.async_copy(packed_hbm.at[idxg.at[b, 0]], rs.at[b], sem_g[b])
            pltpu.async_copy(packed_hbm.at[idxg.at[b, 1]], rd.at[b], sem_g[b])

        def gat_wait(b):
            z = pl.ds(0, _CH)
            pltpu.make_async_copy(packed_hbm.at[z], rs.at[b], sem_g[b]).wait()
            pltpu.make_async_copy(packed_hbm.at[z], rd.at[b], sem_g[b]).wait()

        def sca_issue(b):
            pltpu.async_copy(cs.at[b], acc.at[idxs.at[b, 0]], sem_s[b],
                             add=True)
            pltpu.async_copy(cd.at[b], acc.at[idxs.at[b, 1]], sem_s[b],
                             add=True)

        def sca_wait(b):
            z = pl.ds(0, _CH)
            pltpu.make_async_copy(packed_hbm.at[z], cs.at[b], sem_s[b]).wait()
            pltpu.make_async_copy(packed_hbm.at[z], cd.at[b], sem_s[b]).wait()

        def compute(t, b):
            for r in range(2):
                for g in range(_CH // _L):
                    idxs[b, r, pl.ds(g * _L, _L)] = \
                        idxg[b, r, pl.ds(g * _L, _L)]
            for g in range(_CH // _L):
                row = _iota() + g * _L
                sv = [plsc.load_gather(rs.at[b], [row, _fill(k)])
                      for k in range(8)]
                dv = [plsc.load_gather(rd.at[b], [row, _fill(k)])
                      for k in range(8)]
                el_v = elb[b, pl.ds(g * _L, _L)]
                ei_v = eib[b, pl.ds(g * _L, _L)]
                dx = sv[0] - dv[0]
                dy = sv[1] - dv[1]
                dz = sv[2] - dv[2]
                d2 = dx * dx + dy * dy + dz * dz
                dgt = d2 * _rsqrt(jnp.maximum(d2, 1e-30))
                is_train = (sv[7] + dv[7]) > 0.0
                dtar = jnp.where(is_train, (dgt - el_v) * sv[6], 0.0)
                w = (ei_v - dtar) / el_v
                vx = w * (sv[3] - dv[3])
                vy = w * (sv[4] - dv[4])
                vz = w * (sv[5] - dv[5])
                plsc.store_scatter(cs.at[b], [row, _fill(0)], vx)
                plsc.store_scatter(cs.at[b], [row, _fill(1)], vy)
                plsc.store_scatter(cs.at[b], [row, _fill(2)], vz)
                plsc.store_scatter(cd.at[b], [row, _fill(0)], -vx)
                plsc.store_scatter(cd.at[b], [row, _fill(1)], -vy)
                plsc.store_scatter(cd.at[b], [row, _fill(2)], -vz)

        pltpu.sync_copy(zeros_hbm.at[pl.ds(sid * zrows, zrows)],
                        acc.at[pl.ds(sid * zrows, zrows)])
        zv = jnp.zeros((_L,), jnp.float32)
        for b0 in range(2):
            for g0 in range(_CH // _L):
                r0 = _iota() + g0 * _L
                for k0 in range(3, 8):
                    plsc.store_scatter(cs.at[b0], [r0, _fill(k0)], zv)
                    plsc.store_scatter(cd.at[b0], [r0, _fill(k0)], zv)
        plsc.subcore_barrier()

        lin_issue(0, 0)
        lin_issue(1, 1)
        lin_wait(0)
        gat_issue(0)

        def body(gi, carry):
            for b in range(2):
                t = gi * 2 + b
                b1 = 1 - b
                gat_wait(b)

                @pl.when(t + 1 < nchunks)
                def _():
                    lin_wait(b1)
                    gat_issue(b1)

                @pl.when(t >= 2)
                def _():
                    sca_wait(b)

                compute(t, b)
                sca_issue(b)

                @pl.when(t + 2 < nchunks)
                def _():
                    lin_issue(t + 2, b)
            return carry

        lax.fori_loop(0, nchunks // 2, body, 0)
        sca_wait(0)
        sca_wait(1)
        plsc.subcore_barrier()
        pltpu.sync_copy(acc.at[pl.ds(sid * zrows, zrows)],
                        out_hbm.at[pl.ds(cid * npad + sid * zrows, zrows)])

    return edge


def _make_reduce_kernel(n_real):
    scale = 10.0 / (3.0 * n_real)

    def body(x_ref, o_ref):
        x = x_ref[...]
        s = x[0] + x[1]
        o_ref[0, 0] = jnp.sum(s * s) * scale

    return functools.partial(
        pl.pallas_call,
        body,
        out_shape=jax.ShapeDtypeStruct((1, 1), jnp.float32),
        out_specs=pl.BlockSpec(memory_space=pltpu.SMEM),
    )()


def kernel(edge_inv_global, edge_index, edge_length, a, pos, pos_perturbed,
           node2graph, is_sidechain, log):
    n = pos.shape[0]
    e = edge_index.shape[1]
    g_sz = a.shape[0]

    quant_n = _NW * _L * 2
    npad = ((n + quant_n - 1) // quant_n) * quant_n
    quant_e = _NW * _CH * 2
    epad = ((e + quant_e - 1) // quant_e) * quant_e

    src = jnp.pad(edge_index[0], (0, epad - e))
    dst = jnp.pad(edge_index[1], (0, epad - e))
    el = jnp.pad(edge_length[:, 0], (0, epad - e), constant_values=1.0)
    ei = jnp.pad(edge_inv_global[:, 0], (0, epad - e))

    posp = jnp.pad(pos, ((0, npad - n), (0, 0)))
    qp = jnp.pad(pos_perturbed, ((0, npad - n), (0, 0)))
    n2gp = jnp.pad(node2graph, (0, npad - n))
    sidep = jnp.pad(is_sidechain.astype(jnp.int32), (0, npad - n))
    zeros = jnp.zeros((npad, 8), jnp.float32)

    packed = _make_pack_kernel(npad, g_sz)(a, n2gp, sidep, posp, qp)
    partials = _make_edge_kernel(npad, epad)(src, dst, el, ei, packed, zeros)
    red = _make_reduce_kernel(n)(partials.reshape(_NC, npad * 8 // 128, 128))
    return red[0, 0]


# speedup vs baseline: 202.0216x; 1.0335x over previous
import functools

import jax
import jax.numpy as jnp
from jax import lax
from jax.experimental import pallas as pl
from jax.experimental.pallas import tpu as pltpu
from jax.experimental.pallas import tpu_sc as plsc

_NC = 2
_NS = 16
_NW = _NC * _NS
_L = 16
_CH = 128

_MESH = plsc.VectorSubcoreMesh(
    core_axis_name="c", subcore_axis_name="s", num_cores=_NC, num_subcores=_NS
)

_SC_PARAMS = pltpu.CompilerParams(
    needs_layout_passes=False, use_tc_tiling_on_sc=False
)


def _rsqrt(x):
    i = plsc.bitcast(x, jnp.int32)
    i = jnp.int32(0x5F3759DF) - lax.shift_right_logical(i, 1)
    y = plsc.bitcast(i, jnp.float32)
    for _ in range(3):
        y = y * (1.5 - 0.5 * x * y * y)
    return y


def _iota():
    return jnp.arange(_L, dtype=jnp.int32)


def _fill(k):
    return jnp.full((_L,), k, dtype=jnp.int32)


def _make_pack_kernel(npad, g_sz):
    npt = npad // _NW
    grp = npt // _L

    @functools.partial(
        pl.kernel,
        out_type=jax.ShapeDtypeStruct((npad, 8), jnp.float32),
        mesh=_MESH,
        compiler_params=_SC_PARAMS,
        scratch_types=[
            pltpu.VMEM((g_sz,), jnp.float32),
            pltpu.VMEM((npt,), jnp.int32),
            pltpu.VMEM((npt,), jnp.int32),
            pltpu.VMEM((npt, 3), jnp.float32),
            pltpu.VMEM((npt, 3), jnp.float32),
            pltpu.VMEM((npt, 8), jnp.float32),
        ],
    )
    def pack(a_hbm, n2g_hbm, side_hbm, pos_hbm, q_hbm, out_hbm,
             ctab, n2g, side, posb, qb, packb):
        wid = lax.axis_index("c") * _NS + lax.axis_index("s")
        base = wid * npt
        pltpu.sync_copy(a_hbm, ctab)
        pltpu.sync_copy(n2g_hbm.at[pl.ds(base, npt)], n2g)
        pltpu.sync_copy(side_hbm.at[pl.ds(base, npt)], side)
        pltpu.sync_copy(pos_hbm.at[pl.ds(base, npt)], posb)
        pltpu.sync_copy(q_hbm.at[pl.ds(base, npt)], qb)
        for i in range(g_sz // _L):
            av = ctab[pl.ds(i * _L, _L)]
            r = av / (1.0 - av)
            ctab[pl.ds(i * _L, _L)] = r * _rsqrt(r)

        def body(g, carry):
            row = _iota() + g * _L
            gidx = plsc.load_gather(n2g, [row])
            sidv = plsc.load_gather(side, [row]).astype(jnp.float32)
            cv = plsc.load_gather(ctab, [gidx])
            for k in range(3):
                plsc.store_scatter(packb, [row, _fill(k)],
                                   plsc.load_gather(posb, [row, _fill(k)]))
                plsc.store_scatter(packb, [row, _fill(3 + k)],
                                   plsc.load_gather(qb, [row, _fill(k)]))
            plsc.store_scatter(packb, [row, _fill(6)], cv)
            plsc.store_scatter(packb, [row, _fill(7)], sidv)
            return carry

        lax.fori_loop(0, grp, body, 0)
        pltpu.sync_copy(packb, out_hbm.at[pl.ds(base, npt)])

    return pack


def _make_edge_kernel(npad, epad):
    nchunks_tot = epad // _CH
    nsteps = ((nchunks_tot + _NW - 1) // _NW + 1) // 2 * 2
    zrows = npad // _NS

    assert nsteps >= 4 and nsteps % 2 == 0

    @functools.partial(
        pl.kernel,
        out_type=jax.ShapeDtypeStruct((_NC * npad, 8), jnp.float32),
        mesh=_MESH,
        compiler_params=_SC_PARAMS,
        scratch_types=[
            pltpu.VMEM((2, 2, _CH), jnp.int32),
            pltpu.VMEM((2, 2, _CH), jnp.int32),
            pltpu.VMEM((2, _CH), jnp.float32),
            pltpu.VMEM((2, _CH), jnp.float32),
            pltpu.VMEM((2, _CH, 8), jnp.float32),
            pltpu.VMEM((2, _CH, 8), jnp.float32),
            pltpu.VMEM((2, _CH, 8), jnp.float32),
            pltpu.VMEM((2, _CH, 8), jnp.float32),
            pltpu.VMEM_SHARED((npad, 8), jnp.float32),
            pltpu.SemaphoreType.DMA,
            pltpu.SemaphoreType.DMA,
            pltpu.SemaphoreType.DMA,
            pltpu.SemaphoreType.DMA,
            pltpu.SemaphoreType.DMA,
            pltpu.SemaphoreType.DMA,
        ],
    )
    def edge(eidx_hbm, el_hbm, ei_hbm, packed_hbm, out_hbm,
             idxg, idxs, elb, eib, rs, rd, cs, cd, acc,
             sl0, sl1, sg0, sg1, ss0, ss1):
        cid = lax.axis_index("c")
        sid = lax.axis_index("s")
        wid = cid * _NS + sid
        sem_lin = (sl0, sl1)
        sem_g = (sg0, sg1)
        sem_s = (ss0, ss1)

        def cond(t):
            return (t * _NW + wid) < nchunks_tot

        def ebase(t):
            return pl.multiple_of((t * _NW + wid) * _CH, _CH)

        def lin_issue(t, b):
            eb = ebase(t)
            pltpu.async_copy(eidx_hbm.at[0, pl.ds(eb, _CH)], idxg.at[b, 0],
                             sem_lin[b])
            pltpu.async_copy(eidx_hbm.at[1, pl.ds(eb, _CH)], idxg.at[b, 1],
                             sem_lin[b])
            pltpu.async_copy(el_hbm.at[pl.ds(eb, _CH)], elb.at[b], sem_lin[b])
            pltpu.async_copy(ei_hbm.at[pl.ds(eb, _CH)], eib.at[b], sem_lin[b])

        def lin_wait(b):
            z = pl.ds(0, _CH)
            pltpu.make_async_copy(eidx_hbm.at[0, z], idxg.at[b, 0],
                                  sem_lin[b]).wait()
            pltpu.make_async_copy(eidx_hbm.at[1, z], idxg.at[b, 1],
                                  sem_lin[b]).wait()
            pltpu.make_async_copy(el_hbm.at[z], elb.at[b], sem_lin[b]).wait()
            pltpu.make_async_copy(ei_hbm.at[z], eib.at[b], sem_lin[b]).wait()

        def gat_issue(b):
            pltpu.async_copy(packed_hbm.at[idxg.at[b, 0]], rs.at[b], sem_g[b])
            pltpu.async_copy(packed_hbm.at[idxg.at[b, 1]], rd.at[b], sem_g[b])

        def gat_wait(b):
            z = pl.ds(0, _CH)
            pltpu.make_async_copy(packed_hbm.at[z], rs.at[b], sem_g[b]).wait()
            pltpu.make_async_copy(packed_hbm.at[z], rd.at[b], sem_g[b]).wait()

        def sca_issue(b):
            pltpu.async_copy(cs.at[b], acc.at[idxs.at[b, 0]], sem_s[b],
                             add=True)
            pltpu.async_copy(cd.at[b], acc.at[idxs.at[b, 1]], sem_s[b],
                             add=True)

        def sca_wait(b):
            z = pl.ds(0, _CH)
            pltpu.make_async_copy(packed_hbm.at[z], cs.at[b], sem_s[b]).wait()
            pltpu.make_async_copy(packed_hbm.at[z], cd.at[b], sem_s[b]).wait()

        def compute(t, b):
            for r in range(2):
                for g in range(_CH // _L):
                    idxs[b, r, pl.ds(g * _L, _L)] = \
                        idxg[b, r, pl.ds(g * _L, _L)]
            for g in range(_CH // _L):
                row = _iota() + g * _L
                sv = [plsc.load_gather(rs.at[b], [row, _fill(k)])
                      for k in range(8)]
                dv = [plsc.load_gather(rd.at[b], [row, _fill(k)])
                      for k in range(8)]
                el_v = elb[b, pl.ds(g * _L, _L)]
                ei_v = eib[b, pl.ds(g * _L, _L)]
                dx = sv[0] - dv[0]
                dy = sv[1] - dv[1]
                dz = sv[2] - dv[2]
                d2 = dx * dx + dy * dy + dz * dz
                dgt = d2 * _rsqrt(jnp.maximum(d2, 1e-30))
                is_train = (sv[7] + dv[7]) > 0.0
                dtar = jnp.where(is_train, (dgt - el_v) * sv[6], 0.0)
                w = (ei_v - dtar) / el_v
                vx = w * (sv[3] - dv[3])
                vy = w * (sv[4] - dv[4])
                vz = w * (sv[5] - dv[5])
                plsc.store_scatter(cs.at[b], [row, _fill(0)], vx)
                plsc.store_scatter(cs.at[b], [row, _fill(1)], vy)
                plsc.store_scatter(cs.at[b], [row, _fill(2)], vz)
                plsc.store_scatter(cd.at[b], [row, _fill(0)], -vx)
                plsc.store_scatter(cd.at[b], [row, _fill(1)], -vy)
                plsc.store_scatter(cd.at[b], [row, _fill(2)], -vz)

        zv = jnp.zeros((_L,), jnp.float32)
        for b0 in range(2):
            for g0 in range(_CH // _L):
                r0 = _iota() + g0 * _L
                for k0 in range(8):
                    plsc.store_scatter(cs.at[b0], [r0, _fill(k0)], zv)
                    plsc.store_scatter(cd.at[b0], [r0, _fill(k0)], zv)
        for i in range(zrows // _CH):
            pltpu.sync_copy(cs.at[0],
                            acc.at[pl.ds(sid * zrows + i * _CH, _CH)])
        plsc.subcore_barrier()

        @pl.when(cond(0))
        def _():
            lin_issue(0, 0)

        @pl.when(cond(1))
        def _():
            lin_issue(1, 1)

        @pl.when(cond(0))
        def _():
            lin_wait(0)
            gat_issue(0)

        def body(gi, carry):
            for b in range(2):
                t = gi * 2 + b
                b1 = 1 - b

                @pl.when(cond(t))
                def _():
                    gat_wait(b)

                @pl.when(cond(t + 1))
                def _():
                    lin_wait(b1)
                    gat_issue(b1)

                @pl.when(jnp.logical_and(t >= 2, cond(t - 2)))
                def _():
                    sca_wait(b)

                @pl.when(cond(t))
                def _():
                    compute(t, b)
                    sca_issue(b)

                @pl.when(cond(t + 2))
                def _():
                    lin_issue(t + 2, b)
            return carry

        lax.fori_loop(0, nsteps // 2, body, 0)

        @pl.when(cond(nsteps - 2))
        def _():
            sca_wait(0)

        @pl.when(cond(nsteps - 1))
        def _():
            sca_wait(1)

        plsc.subcore_barrier()
        pltpu.sync_copy(acc.at[pl.ds(sid * zrows, zrows)],
                        out_hbm.at[pl.ds(cid * npad + sid * zrows, zrows)])

    return edge


def _make_reduce_kernel(n_real):
    scale = 10.0 / (3.0 * n_real)

    def body(x_ref, o_ref):
        x = x_ref[...]
        s = x[0] + x[1]
        o_ref[0, 0] = jnp.sum(s * s) * scale

    return functools.partial(
        pl.pallas_call,
        body,
        out_shape=jax.ShapeDtypeStruct((1, 1), jnp.float32),
        out_specs=pl.BlockSpec(memory_space=pltpu.SMEM),
    )()


def kernel(edge_inv_global, edge_index, edge_length, a, pos, pos_perturbed,
           node2graph, is_sidechain, log):
    n = pos.shape[0]
    e = edge_index.shape[1]
    g_sz = a.shape[0]

    quant_n = _NS * _CH
    npad = ((n + quant_n - 1) // quant_n) * quant_n
    epad = ((e + _CH - 1) // _CH) * _CH

    eidx = edge_index
    el = edge_length.reshape(-1)
    ei = edge_inv_global.reshape(-1)
    if epad != e:
        eidx = jnp.pad(eidx, ((0, 0), (0, epad - e)))
        el = jnp.pad(el, (0, epad - e), constant_values=1.0)
        ei = jnp.pad(ei, (0, epad - e))

    posp = jnp.pad(pos, ((0, npad - n), (0, 0)))
    qp = jnp.pad(pos_perturbed, ((0, npad - n), (0, 0)))
    n2gp = jnp.pad(node2graph, (0, npad - n))
    sidep = jnp.pad(is_sidechain.astype(jnp.int32), (0, npad - n))

    packed = _make_pack_kernel(npad, g_sz)(a, n2gp, sidep, posp, qp)
    partials = _make_edge_kernel(npad, epad)(eidx, el, ei, packed)
    red = _make_reduce_kernel(n)(partials.reshape(_NC, npad * 8 // 128, 128))
    return red[0, 0]
